# Initial kernel scaffold; baseline (speedup 1.0000x reference)
import jax
import jax.numpy as jnp
from jax.experimental import pallas as pl


def kernel(state, edge_index, edge_attr, phi_W, phi_b, rho_W, rho_b, readin_W, readin_b, taps1_W, taps1_b, taps2_W, taps2_b, readout_W, readout_b):
    raise NotImplementedError("write your pallas kernel here")


# speedup vs baseline: 8.0139x; 8.0139x over previous
import functools

import jax
import jax.numpy as jnp
from jax import lax
from jax.experimental import pallas as pl
from jax.experimental.pallas import tpu as pltpu
from jax.experimental.pallas import tpu_sc as plsc

F = 16
NC_TOTAL = 32
SUB = 128
NSUBCH = 16
EB = NSUBCH * SUB
NSUB = 16
LANES = 16

_SPLAT_DNUMS = lax.GatherDimensionNumbers(
    offset_dims=(), collapsed_slice_dims=(0,), start_index_map=(0,))


def _lane_splat(v, lane):
    idx = jnp.full((LANES, 1), lane, jnp.int32)
    return lax.gather(v, idx, _SPLAT_DNUMS, (1,),
                      mode=lax.GatherScatterMode.PROMISE_IN_BOUNDS)


def _prop_sc(x_flat, src2, dst_r, w_r, n, np_, bpt, rpt):
    n_acc = np_ + LANES
    mesh = plsc.VectorSubcoreMesh(core_axis_name="c", subcore_axis_name="s")

    @functools.partial(
        pl.kernel,
        mesh=mesh,
        compiler_params=pltpu.CompilerParams(use_tc_tiling_on_sc=False),
        out_type=jax.ShapeDtypeStruct((2 * np_, F), jnp.float32),
        scratch_types=[
            pltpu.VMEM((NSUBCH, SUB), jnp.int32),
            pltpu.VMEM((NSUBCH, SUB), jnp.int32),
            pltpu.VMEM((NSUBCH, SUB), jnp.float32),
            pltpu.VMEM((SUB, F), jnp.float32),
            pltpu.VMEM_SHARED((n_acc, F), jnp.float32),
            pltpu.SemaphoreType.DMA,
        ],
    )
    def k(x_hbm, src_hbm, dst_hbm, w_hbm, y_hbm, ts_src, ts_dst, ts_w, rows,
          acc, sem):
        c = lax.axis_index("c")
        s = lax.axis_index("s")

        def _zrow(e, _):
            rows[e, :] = jnp.zeros((F,), jnp.float32)
            return 0
        lax.fori_loop(0, SUB, _zrow, 0)
        base = s * rpt
        nfull = rpt // SUB
        rem = rpt - nfull * SUB

        def _zcp(r, _):
            pltpu.sync_copy(rows, acc.at[pl.ds(base + r * SUB, SUB)])
            return 0
        lax.fori_loop(0, nfull, _zcp, 0)
        if rem:
            pltpu.sync_copy(rows.at[pl.ds(0, rem)],
                            acc.at[pl.ds(base + nfull * SUB, rem)])
        plsc.subcore_barrier()

        def blk_body(b, _):
            pltpu.sync_copy(src_hbm.at[c, s, b], ts_src)
            pltpu.sync_copy(dst_hbm.at[s, b], ts_dst)
            pltpu.sync_copy(w_hbm.at[s, b], ts_w)

            def sub_body(j, _):
                pltpu.async_copy(x_hbm.at[ts_src.at[j]], rows, sem).wait()

                def g_body(g, _):
                    wv = ts_w[j, pl.ds(g * LANES, LANES)]
                    e0 = g * LANES
                    for l in range(LANES):
                        spl = _lane_splat(wv, l)
                        rows[e0 + l, :] = rows[e0 + l, :] * spl
                    return 0
                lax.fori_loop(0, SUB // LANES, g_body, 0)
                pltpu.sync_copy(rows, acc.at[ts_dst.at[j]], add=True)
                return 0
            lax.fori_loop(0, NSUBCH, sub_body, 0)
            return 0
        lax.fori_loop(0, bpt, blk_body, 0)
        plsc.subcore_barrier()

        pltpu.sync_copy(acc.at[pl.ds(base, rpt)],
                        y_hbm.at[pl.ds(c * np_ + base, rpt)])

    return k(x_flat, src2, dst_r, w_r)


def _readin_tc(state, big_w, big_b, n, np_, bn):
    def body(s_ref, w_ref, b_ref, o_ref):
        hb = jnp.dot(s_ref[...], w_ref[...],
                     preferred_element_type=jnp.float32) + b_ref[...]
        o_ref[0] = hb[:, :F]
        o_ref[1] = hb[:, F:]

    return pl.pallas_call(
        body,
        grid=(n // bn,),
        in_specs=[
            pl.BlockSpec((bn, state.shape[1]), lambda i: (i, 0)),
            pl.BlockSpec(big_w.shape, lambda i: (0, 0)),
            pl.BlockSpec(big_b.shape, lambda i: (0, 0)),
        ],
        out_specs=pl.BlockSpec((2, bn, F), lambda i: (0, i, 0)),
        out_shape=jax.ShapeDtypeStruct((2, np_, F), jnp.float32),
    )(state, big_w, big_b)


def _combine_tc(h2, y1, y2, y3, taps_w, taps_b, n, np_, bn):
    def body(h_ref, y1_ref, y2_ref, y3_ref, w_ref, b_ref, o_ref):
        def cat(r):
            v = r[...]
            return jnp.concatenate([v[0], v[1]], axis=-1)
        acc = (jnp.dot(cat(h_ref), w_ref[0], preferred_element_type=jnp.float32)
               + jnp.dot(cat(y1_ref), w_ref[1], preferred_element_type=jnp.float32)
               + jnp.dot(cat(y2_ref), w_ref[2], preferred_element_type=jnp.float32)
               + jnp.dot(cat(y3_ref), w_ref[3], preferred_element_type=jnp.float32)
               + b_ref[...])
        r = jnp.where(acc >= 0, acc, 0.01 * acc)
        o_ref[0] = r[:, :F]
        o_ref[1] = r[:, F:]

    spec2 = pl.BlockSpec((2, bn, F), lambda i: (0, i, 0))
    return pl.pallas_call(
        body,
        grid=(n // bn,),
        in_specs=[
            spec2, spec2, spec2, spec2,
            pl.BlockSpec(taps_w.shape, lambda i: (0, 0, 0)),
            pl.BlockSpec(taps_b.shape, lambda i: (0, 0)),
        ],
        out_specs=spec2,
        out_shape=jax.ShapeDtypeStruct((2, np_, F), jnp.float32),
    )(h2, y1, y2, y3, taps_w, taps_b)


def _combine_readout_tc(h2, y1, y2, y3, taps_w, taps_b, ro_w, ro_b, n, bn):
    def body(h_ref, y1_ref, y2_ref, y3_ref, w_ref, b_ref, rw_ref, rb_ref,
             o_ref):
        def cat(r):
            v = r[...]
            return jnp.concatenate([v[0], v[1]], axis=-1)
        acc = (jnp.dot(cat(h_ref), w_ref[0], preferred_element_type=jnp.float32)
               + jnp.dot(cat(y1_ref), w_ref[1], preferred_element_type=jnp.float32)
               + jnp.dot(cat(y2_ref), w_ref[2], preferred_element_type=jnp.float32)
               + jnp.dot(cat(y3_ref), w_ref[3], preferred_element_type=jnp.float32)
               + b_ref[...])
        r = jnp.where(acc >= 0, acc, 0.01 * acc)
        o_ref[...] = jnp.dot(r, rw_ref[...],
                             preferred_element_type=jnp.float32) + rb_ref[...]

    spec2 = pl.BlockSpec((2, bn, F), lambda i: (0, i, 0))
    out_dim = ro_w.shape[1]
    return pl.pallas_call(
        body,
        grid=(n // bn,),
        in_specs=[
            spec2, spec2, spec2, spec2,
            pl.BlockSpec(taps_w.shape, lambda i: (0, 0, 0)),
            pl.BlockSpec(taps_b.shape, lambda i: (0, 0)),
            pl.BlockSpec(ro_w.shape, lambda i: (0, 0)),
            pl.BlockSpec(ro_b.shape, lambda i: (0, 0)),
        ],
        out_specs=pl.BlockSpec((bn, out_dim), lambda i: (i, 0)),
        out_shape=jax.ShapeDtypeStruct((n, out_dim), jnp.float32),
    )(h2, y1, y2, y3, taps_w, taps_b, ro_w, ro_b)


def kernel(state, edge_index, edge_attr, phi_W, phi_b, rho_W, rho_b,
           readin_W, readin_b, taps1_W, taps1_b, taps2_W, taps2_b,
           readout_W, readout_b):
    n = state.shape[0]
    e = edge_index.shape[1]
    state_dim = state.shape[1]
    n_obs_pairs = (state_dim - 10) // 2

    tail_w = (phi_W @ rho_W) @ readin_W[10:12]
    big_w = jnp.concatenate(
        [readin_W[:10], jnp.tile(tail_w, (n_obs_pairs, 1))], axis=0)
    big_b = ((n_obs_pairs * phi_b) @ rho_W + rho_b) @ readin_W[10:12] + readin_b
    big_b = big_b.reshape(1, NC_TOTAL)

    per_tile = -(-e // NSUB)
    bpt = -(-per_tile // EB)
    e_pad = NSUB * bpt * EB
    pad = e_pad - e
    src = edge_index[0]
    dst = edge_index[1]
    w = edge_attr[:, 0]
    rpt = ((-(-n // NSUB)) + 7) // 8 * 8
    np_ = NSUB * rpt
    src_p = jnp.concatenate([src, jnp.zeros((pad,), jnp.int32)])
    dst_p = jnp.concatenate([dst, jnp.full((pad,), n, jnp.int32)])
    w_p = jnp.concatenate([w, jnp.zeros((pad,), jnp.float32)])
    src2 = jnp.stack([src_p, src_p + np_]).reshape(2, NSUB, bpt, NSUBCH, SUB)
    dst_r = dst_p.reshape(NSUB, bpt, NSUBCH, SUB)
    w_r = w_p.reshape(NSUB, bpt, NSUBCH, SUB)

    bn = 4000
    assert n % bn == 0

    h2 = _readin_tc(state, big_w, big_b, n, np_, bn)
    h_flat = h2.reshape(2 * np_, F)

    b1 = taps1_b.reshape(1, NC_TOTAL)
    b2 = taps2_b.reshape(1, NC_TOTAL)
    ro_b = readout_b.reshape(1, readout_W.shape[1])

    y1 = _prop_sc(h_flat, src2, dst_r, w_r, n, np_, bpt, rpt)
    y2 = _prop_sc(y1, src2, dst_r, w_r, n, np_, bpt, rpt)
    y3 = _prop_sc(y2, src2, dst_r, w_r, n, np_, bpt, rpt)
    h2 = _combine_tc(h2, y1.reshape(2, np_, F), y2.reshape(2, np_, F),
                     y3.reshape(2, np_, F), taps1_W, b1, n, np_, bn)
    h_flat = h2.reshape(2 * np_, F)

    y1 = _prop_sc(h_flat, src2, dst_r, w_r, n, np_, bpt, rpt)
    y2 = _prop_sc(y1, src2, dst_r, w_r, n, np_, bpt, rpt)
    y3 = _prop_sc(y2, src2, dst_r, w_r, n, np_, bpt, rpt)
    out = _combine_readout_tc(h2, y1.reshape(2, np_, F), y2.reshape(2, np_, F),
                              y3.reshape(2, np_, F), taps2_W, b2, readout_W,
                              ro_b, n, bn)
    return out


# speedup vs baseline: 18.7529x; 2.3401x over previous
import functools

import jax
import jax.numpy as jnp
from jax import lax
from jax.experimental import pallas as pl
from jax.experimental.pallas import tpu as pltpu
from jax.experimental.pallas import tpu_sc as plsc

F = 16
NC_TOTAL = 32
SUB = 128
NSUBCH = 4
EB = NSUBCH * SUB
NSUB = 16
LANES = 16

_SPLAT_DNUMS = lax.GatherDimensionNumbers(
    offset_dims=(), collapsed_slice_dims=(0,), start_index_map=(0,))


def _lane_splat(v, lane):
    idx = jnp.full((LANES, 1), lane, jnp.int32)
    return lax.gather(v, idx, _SPLAT_DNUMS, (1,),
                      mode=lax.GatherScatterMode.PROMISE_IN_BOUNDS)


def _prop_sc(x_flat, src2, dst_r, w_r, n, np_, bpt, rpt):
    n_acc = np_
    mesh = plsc.VectorSubcoreMesh(core_axis_name="c", subcore_axis_name="s")

    @functools.partial(
        pl.kernel,
        mesh=mesh,
        compiler_params=pltpu.CompilerParams(use_tc_tiling_on_sc=False),
        out_type=jax.ShapeDtypeStruct((2 * np_, F), jnp.float32),
        scratch_types=[
            pltpu.VMEM((3, NSUBCH, SUB), jnp.int32),
            pltpu.VMEM((3, NSUBCH, SUB), jnp.int32),
            pltpu.VMEM((3, NSUBCH, SUB), jnp.float32),
            pltpu.VMEM((2, NSUBCH * SUB, F), jnp.float32),
            pltpu.VMEM_SHARED((n_acc, F), jnp.float32),
            pltpu.SemaphoreType.DMA,
            pltpu.SemaphoreType.DMA,
            pltpu.SemaphoreType.DMA,
        ],
    )
    def k(x_hbm, src_hbm, dst_hbm, w_hbm, y_hbm, ts_src, ts_dst, ts_w, rows,
          acc, g_sem, s_sem, i_sem):
        c = lax.axis_index("c")
        s = lax.axis_index("s")

        def rslot(p, j):
            return rows.at[p].at[pl.ds(j * SUB, SUB)]

        def issue_idx(b, p3):
            pltpu.async_copy(src_hbm.at[c, s, b], ts_src.at[p3], i_sem)
            pltpu.async_copy(dst_hbm.at[s, b], ts_dst.at[p3], i_sem)
            pltpu.async_copy(w_hbm.at[s, b], ts_w.at[p3], i_sem)

        def wait_idx(p3):
            for ref in (ts_src.at[p3], ts_dst.at[p3]):
                pltpu.make_async_copy(dst_hbm.at[s, 0], ref, i_sem).wait()
            pltpu.make_async_copy(w_hbm.at[s, 0], ts_w.at[p3], i_sem).wait()

        def issue_gathers(p2, p3):
            for j in range(NSUBCH):
                pltpu.async_copy(x_hbm.at[ts_src.at[p3, j]], rslot(p2, j),
                                 g_sem)

        def drain16(p2, sem):
            for j in range(NSUBCH):
                pltpu.make_async_copy(x_hbm.at[pl.ds(0, SUB)], rslot(p2, j),
                                      sem).wait()

        def scale_scatter(p2, p3):
            for j in range(NSUBCH):
                slot = rslot(p2, j)

                def g_body(g, _):
                    e0 = g * LANES
                    wv = ts_w[p3, j, pl.ds(e0, LANES)]
                    for l in range(LANES):
                        spl = _lane_splat(wv, l)
                        slot[e0 + l, :] = slot[e0 + l, :] * spl
                    return 0
                lax.fori_loop(0, SUB // LANES, g_body, 0, unroll=2)
                pltpu.async_copy(slot, acc.at[ts_dst.at[p3, j]], s_sem,
                                 add=True)

        def _zrow(e, _):
            rows[0, e, :] = jnp.zeros((F,), jnp.float32)
            return 0
        lax.fori_loop(0, SUB, _zrow, 0)
        base = s * rpt
        nfull = rpt // SUB
        rem = rpt - nfull * SUB

        def _zcp(r, _):
            pltpu.sync_copy(rows.at[0].at[pl.ds(0, SUB)],
                            acc.at[pl.ds(base + r * SUB, SUB)])
            return 0
        lax.fori_loop(0, nfull, _zcp, 0)
        if rem:
            pltpu.sync_copy(rows.at[0].at[pl.ds(0, rem)],
                            acc.at[pl.ds(base + nfull * SUB, rem)])
        plsc.subcore_barrier()

        issue_idx(0, 0)
        wait_idx(0)
        if bpt > 1:
            issue_idx(1, 1)
        issue_gathers(0, 0)

        def blk_body(b, _):
            p2 = lax.rem(b, 2)
            q2 = lax.rem(b + 1, 2)
            p3 = lax.rem(b, 3)
            drain16(p2, g_sem)

            @pl.when(b < bpt - 1)
            def _():
                q3 = lax.rem(b + 1, 3)

                @pl.when(b >= 1)
                def _():
                    drain16(q2, s_sem)
                wait_idx(q3)
                issue_gathers(q2, q3)

                @pl.when(b < bpt - 2)
                def _():
                    issue_idx(b + 2, lax.rem(b + 2, 3))
            scale_scatter(p2, p3)
            return 0
        lax.fori_loop(0, bpt, blk_body, 0)
        if bpt > 1:
            drain16((bpt - 2) % 2, s_sem)
        drain16((bpt - 1) % 2, s_sem)
        plsc.subcore_barrier()

        pltpu.sync_copy(acc.at[pl.ds(base, rpt)],
                        y_hbm.at[pl.ds(c * np_ + base, rpt)])

    return k(x_flat, src2, dst_r, w_r)


def _readin_tc(state, big_w, big_b, n, np_, bn):
    def body(s_ref, w_ref, b_ref, o_ref):
        hb = jnp.dot(s_ref[...], w_ref[...],
                     preferred_element_type=jnp.float32) + b_ref[...]
        o_ref[0] = hb[:, :F]
        o_ref[1] = hb[:, F:]

    return pl.pallas_call(
        body,
        grid=(n // bn,),
        in_specs=[
            pl.BlockSpec((bn, state.shape[1]), lambda i: (i, 0)),
            pl.BlockSpec(big_w.shape, lambda i: (0, 0)),
            pl.BlockSpec(big_b.shape, lambda i: (0, 0)),
        ],
        out_specs=pl.BlockSpec((2, bn, F), lambda i: (0, i, 0)),
        out_shape=jax.ShapeDtypeStruct((2, np_, F), jnp.float32),
    )(state, big_w, big_b)


def _combine_tc(h2, y1, y2, y3, taps_w, taps_b, n, np_, bn):
    def body(h_ref, y1_ref, y2_ref, y3_ref, w_ref, b_ref, o_ref):
        def cat(r):
            v = r[...]
            return jnp.concatenate([v[0], v[1]], axis=-1)
        acc = (jnp.dot(cat(h_ref), w_ref[0], preferred_element_type=jnp.float32)
               + jnp.dot(cat(y1_ref), w_ref[1], preferred_element_type=jnp.float32)
               + jnp.dot(cat(y2_ref), w_ref[2], preferred_element_type=jnp.float32)
               + jnp.dot(cat(y3_ref), w_ref[3], preferred_element_type=jnp.float32)
               + b_ref[...])
        r = jnp.where(acc >= 0, acc, 0.01 * acc)
        o_ref[0] = r[:, :F]
        o_ref[1] = r[:, F:]

    spec2 = pl.BlockSpec((2, bn, F), lambda i: (0, i, 0))
    return pl.pallas_call(
        body,
        grid=(n // bn,),
        in_specs=[
            spec2, spec2, spec2, spec2,
            pl.BlockSpec(taps_w.shape, lambda i: (0, 0, 0)),
            pl.BlockSpec(taps_b.shape, lambda i: (0, 0)),
        ],
        out_specs=spec2,
        out_shape=jax.ShapeDtypeStruct((2, np_, F), jnp.float32),
    )(h2, y1, y2, y3, taps_w, taps_b)


def _combine_readout_tc(h2, y1, y2, y3, taps_w, taps_b, ro_w, ro_b, n, bn):
    def body(h_ref, y1_ref, y2_ref, y3_ref, w_ref, b_ref, rw_ref, rb_ref,
             o_ref):
        def cat(r):
            v = r[...]
            return jnp.concatenate([v[0], v[1]], axis=-1)
        acc = (jnp.dot(cat(h_ref), w_ref[0], preferred_element_type=jnp.float32)
               + jnp.dot(cat(y1_ref), w_ref[1], preferred_element_type=jnp.float32)
               + jnp.dot(cat(y2_ref), w_ref[2], preferred_element_type=jnp.float32)
               + jnp.dot(cat(y3_ref), w_ref[3], preferred_element_type=jnp.float32)
               + b_ref[...])
        r = jnp.where(acc >= 0, acc, 0.01 * acc)
        o_ref[...] = jnp.dot(r, rw_ref[...],
                             preferred_element_type=jnp.float32) + rb_ref[...]

    spec2 = pl.BlockSpec((2, bn, F), lambda i: (0, i, 0))
    out_dim = ro_w.shape[1]
    return pl.pallas_call(
        body,
        grid=(n // bn,),
        in_specs=[
            spec2, spec2, spec2, spec2,
            pl.BlockSpec(taps_w.shape, lambda i: (0, 0, 0)),
            pl.BlockSpec(taps_b.shape, lambda i: (0, 0)),
            pl.BlockSpec(ro_w.shape, lambda i: (0, 0)),
            pl.BlockSpec(ro_b.shape, lambda i: (0, 0)),
        ],
        out_specs=pl.BlockSpec((bn, out_dim), lambda i: (i, 0)),
        out_shape=jax.ShapeDtypeStruct((n, out_dim), jnp.float32),
    )(h2, y1, y2, y3, taps_w, taps_b, ro_w, ro_b)


def kernel(state, edge_index, edge_attr, phi_W, phi_b, rho_W, rho_b,
           readin_W, readin_b, taps1_W, taps1_b, taps2_W, taps2_b,
           readout_W, readout_b):
    n = state.shape[0]
    e = edge_index.shape[1]
    state_dim = state.shape[1]
    n_obs_pairs = (state_dim - 10) // 2

    tail_w = (phi_W @ rho_W) @ readin_W[10:12]
    big_w = jnp.concatenate(
        [readin_W[:10], jnp.tile(tail_w, (n_obs_pairs, 1))], axis=0)
    big_b = ((n_obs_pairs * phi_b) @ rho_W + rho_b) @ readin_W[10:12] + readin_b
    big_b = big_b.reshape(1, NC_TOTAL)

    per_tile = -(-e // NSUB)
    bpt = -(-per_tile // EB)
    e_pad = NSUB * bpt * EB
    pad = e_pad - e
    src = edge_index[0]
    dst = edge_index[1]
    w = edge_attr[:, 0]
    rpt = ((-(-n // NSUB)) + 7) // 8 * 8
    np_ = NSUB * rpt
    src_p = jnp.concatenate([src, jnp.zeros((pad,), jnp.int32)])
    dst_p = jnp.concatenate([dst, jnp.zeros((pad,), jnp.int32)])
    w_p = jnp.concatenate([w, jnp.zeros((pad,), jnp.float32)])
    src2 = jnp.stack([src_p, src_p + np_]).reshape(2, NSUB, bpt, NSUBCH, SUB)
    dst_r = dst_p.reshape(NSUB, bpt, NSUBCH, SUB)
    w_r = w_p.reshape(NSUB, bpt, NSUBCH, SUB)

    bn = 4000
    assert n % bn == 0

    h2 = _readin_tc(state, big_w, big_b, n, np_, bn)
    h_flat = h2.reshape(2 * np_, F)

    b1 = taps1_b.reshape(1, NC_TOTAL)
    b2 = taps2_b.reshape(1, NC_TOTAL)
    ro_b = readout_b.reshape(1, readout_W.shape[1])

    y1 = _prop_sc(h_flat, src2, dst_r, w_r, n, np_, bpt, rpt)
    y2 = _prop_sc(y1, src2, dst_r, w_r, n, np_, bpt, rpt)
    y3 = _prop_sc(y2, src2, dst_r, w_r, n, np_, bpt, rpt)
    h2 = _combine_tc(h2, y1.reshape(2, np_, F), y2.reshape(2, np_, F),
                     y3.reshape(2, np_, F), taps1_W, b1, n, np_, bn)
    h_flat = h2.reshape(2 * np_, F)

    y1 = _prop_sc(h_flat, src2, dst_r, w_r, n, np_, bpt, rpt)
    y2 = _prop_sc(y1, src2, dst_r, w_r, n, np_, bpt, rpt)
    y3 = _prop_sc(y2, src2, dst_r, w_r, n, np_, bpt, rpt)
    out = _combine_readout_tc(h2, y1.reshape(2, np_, F), y2.reshape(2, np_, F),
                              y3.reshape(2, np_, F), taps2_W, b2, readout_W,
                              ro_b, n, bn)
    return out
